# baseline (speedup 1.0000x reference)
import functools

import jax
import jax.numpy as jnp
import numpy as np
from jax.experimental import pallas as pl
from jax.experimental.pallas import tpu as pltpu
from jax.experimental.pallas import tpu_sc as plsc


def _tc_table_kernel(s_ref, e_ref, f_ref, tab_ref, *, d, t_len, tb, n_half):
    t = pl.program_id(1)
    nt = t_len // tb

    @pl.when(t < nt)
    def _():
        tab_ref[...] = f_ref[0, :, pl.ds(t * tb, tb)].T

    @pl.when(t >= nt)
    def _():
        s = s_ref[0, :, pl.ds((t - nt) * n_half, n_half)]
        e = e_ref[0, :, pl.ds((t - nt) * n_half, n_half)]
        col = jax.lax.broadcasted_iota(jnp.int32, (t_len, n_half), 0)
        in_span = (col >= s) & (col < e)
        inv_len = 1.0 / (e - s).astype(jnp.float32)
        m_avg = jnp.where(in_span, inv_len, 0.0).astype(jnp.bfloat16)
        dn = (((0,), (1,)), ((), ()))
        tab_ref[...] = jax.lax.dot_general(
            m_avg,
            f_ref[0].astype(jnp.bfloat16),
            dn,
            preferred_element_type=jnp.float32,
        )


def _sc_gather(tab_half, idx, out_rows, hw):
    mesh = plsc.VectorSubcoreMesh(core_axis_name="core", subcore_axis_name="subcore")

    @pl.kernel(
        out_type=jax.ShapeDtypeStruct((idx.shape[1], hw), jnp.float32), mesh=mesh
    )
    def k(tab_hbm, i_hbm, o_hbm):
        def body(i_v, o_v):
            pltpu.sync_copy(tab_hbm.at[i_v.at[0]], o_v)

        pltpu.emit_pipeline(
            body,
            grid=(idx.shape[1] // out_rows,),
            in_specs=[pl.BlockSpec((1, out_rows), lambda i: (0, i))],
            out_specs=[pl.BlockSpec((out_rows, hw), lambda i: (i, 0))],
            core_axis_name=("core", "subcore"),
            dimension_semantics=(pltpu.PARALLEL,),
        )(i_hbm, o_hbm)

    return k(tab_half, idx)


@jax.jit
def kernel(features, tois):
    b, d, t_len = features.shape
    n = tois.shape[1]
    tb = 512
    n_half = 512
    nt = t_len // tb
    na = n // n_half
    s = tois[:, :, 0]
    e = tois[:, :, 1]

    n_rows = b * t_len + b * n

    def tab_index(i, j):
        return (jnp.where(j < nt, i * nt + j, b * nt + i * na + (j - nt)), 0)

    tab = pl.pallas_call(
        functools.partial(
            _tc_table_kernel, d=d, t_len=t_len, tb=tb, n_half=n_half
        ),
        grid=(b, nt + na),
        in_specs=[
            pl.BlockSpec((1, 1, n), lambda i, j: (i, 0, 0)),
            pl.BlockSpec((1, 1, n), lambda i, j: (i, 0, 0)),
            pl.BlockSpec((1, d, t_len), lambda i, j: (i, 0, 0)),
        ],
        out_specs=pl.BlockSpec((tb, d), tab_index),
        out_shape=jax.ShapeDtypeStruct((n_rows, d), jnp.float32),
    )(s.reshape(b, 1, n), e.reshape(b, 1, n), features)

    base = jnp.arange(b, dtype=jnp.int32)[:, None] * np.int32(t_len)
    r0 = (base + s).reshape(-1)
    r1 = np.int32(b * t_len) + jnp.arange(b * n, dtype=jnp.int32)
    r2 = (base + e - 1).reshape(-1)
    idx = jnp.stack(
        [2 * r0, 2 * r0 + 1, 2 * r1, 2 * r1 + 1, 2 * r2, 2 * r2 + 1], axis=-1
    ).reshape(1, -1)

    hw = d // 2
    out = _sc_gather(tab.reshape(2 * n_rows, hw), idx, 128, hw)
    offsets = jnp.arange(1, b + 1, dtype=jnp.int32) * np.int32(n)
    return out.reshape(b * n, 3 * d), offsets


# speedup vs baseline: 1.4557x; 1.4557x over previous
import functools

import jax
import jax.numpy as jnp
import numpy as np
from jax.experimental import pallas as pl
from jax.experimental.pallas import tpu as pltpu
from jax.experimental.pallas import tpu_sc as plsc


def _tc_table_kernel(s_ref, e_ref, f_ref, tab_ref, *, d, t_len, tb, n_half):
    j = pl.program_id(1)
    nt = t_len // tb
    dh = d // 2

    def avg_half(h, c0):
        s = s_ref[0, :, pl.ds(h * n_half, n_half)]
        e = e_ref[0, :, pl.ds(h * n_half, n_half)]
        col = jax.lax.broadcasted_iota(jnp.int32, (t_len, n_half), 0)
        in_span = (col >= s) & (col < e)
        inv_len = 1.0 / (e - s).astype(jnp.float32)
        m_avg = jnp.where(in_span, inv_len, 0.0).astype(jnp.bfloat16)
        dn = (((0,), (1,)), ((), ()))
        tab_ref[...] = jax.lax.dot_general(
            m_avg,
            f_ref[0, pl.ds(c0, dh), :].astype(jnp.bfloat16),
            dn,
            preferred_element_type=jnp.float32,
        )

    @pl.when(j < nt)
    def _():
        tab_ref[...] = f_ref[0, 0:dh, pl.ds(j * tb, tb)].T

    @pl.when((j >= nt) & (j < 2 * nt))
    def _():
        tab_ref[...] = f_ref[0, dh : 2 * dh, pl.ds((j - nt) * tb, tb)].T

    @pl.when((j >= 2 * nt) & (j < 2 * nt + 2))
    def _():
        avg_half(j - 2 * nt, 0)

    @pl.when(j >= 2 * nt + 2)
    def _():
        avg_half(j - 2 * nt - 2, dh)


def _sc_gather(tab_half, idx, out_shape, blk_r, blk_c):
    mesh = plsc.VectorSubcoreMesh(core_axis_name="core", subcore_axis_name="subcore")
    n_r = out_shape[0] // blk_r
    n_c = out_shape[1] // blk_c

    @pl.kernel(out_type=jax.ShapeDtypeStruct(out_shape, jnp.float32), mesh=mesh)
    def k(tab_hbm, i_hbm, o_hbm):
        def body(i_v, o_v):
            pltpu.sync_copy(tab_hbm.at[i_v.at[0]], o_v)

        pltpu.emit_pipeline(
            body,
            grid=(n_r, n_c),
            in_specs=[pl.BlockSpec((1, blk_r), lambda i, j: (0, j * n_r + i))],
            out_specs=[pl.BlockSpec((blk_r, blk_c), lambda i, j: (i, j))],
            core_axis_name=("core", "subcore"),
            dimension_semantics=(pltpu.PARALLEL, pltpu.PARALLEL),
        )(i_hbm, o_hbm)

    return k(tab_half, idx)


@jax.jit
def kernel(features, tois):
    b, d, t_len = features.shape
    n = tois.shape[1]
    tb = 512
    n_half = 512
    nt = t_len // tb
    na = n // n_half
    s = tois[:, :, 0]
    e = tois[:, :, 1]

    half_rows = b * t_len + b * n

    def tab_index(i, j):
        tr_l = i * nt + j
        tr_r = half_rows // tb + i * nt + (j - nt)
        av_l = b * nt + i * na + (j - 2 * nt)
        av_r = half_rows // tb + b * nt + i * na + (j - 2 * nt - 2)
        return (
            jnp.where(
                j < nt,
                tr_l,
                jnp.where(j < 2 * nt, tr_r, jnp.where(j < 2 * nt + 2, av_l, av_r)),
            ),
            0,
        )

    tab = pl.pallas_call(
        functools.partial(
            _tc_table_kernel, d=d, t_len=t_len, tb=tb, n_half=n_half
        ),
        grid=(b, 2 * (nt + na)),
        in_specs=[
            pl.BlockSpec((1, 1, n), lambda i, j: (i, 0, 0)),
            pl.BlockSpec((1, 1, n), lambda i, j: (i, 0, 0)),
            pl.BlockSpec((1, d, t_len), lambda i, j: (i, 0, 0)),
        ],
        out_specs=pl.BlockSpec((tb, d // 2), tab_index),
        out_shape=jax.ShapeDtypeStruct((2 * half_rows, d // 2), jnp.float32),
    )(s.reshape(b, 1, n), e.reshape(b, 1, n), features)

    base = jnp.arange(b, dtype=jnp.int32)[:, None] * np.int32(t_len)
    r0 = (base + s).reshape(-1)
    r1 = np.int32(b * t_len) + jnp.arange(b * n, dtype=jnp.int32)
    r2 = (base + e - 1).reshape(-1)
    hr = np.int32(half_rows)
    idx = jnp.concatenate(
        [r0, r0 + hr, r1, r1 + hr, r2, r2 + hr]
    ).reshape(1, -1)

    out = _sc_gather(tab, idx, (b * n, 3 * d), 128, d // 2)
    offsets = jnp.arange(1, b + 1, dtype=jnp.int32) * np.int32(n)
    return out, offsets


# speedup vs baseline: 2.6360x; 1.8107x over previous
import functools

import jax
import jax.numpy as jnp
import numpy as np
from jax.experimental import pallas as pl
from jax.experimental.pallas import tpu as pltpu


def _toi_tc_kernel(s_ref, e_ref, f_ref, o_ref, *, d: int, t_len: int, nh: int):
    h = pl.program_id(1)
    f = f_ref[0]
    s = s_ref[0, :, pl.ds(h * nh, nh)]
    e = e_ref[0, :, pl.ds(h * nh, nh)]
    col = jax.lax.broadcasted_iota(jnp.int32, (t_len, nh), 0)
    in_span = (col >= s) & (col < e)
    inv_len = 1.0 / (e - s).astype(jnp.float32)
    fb = f.astype(jnp.bfloat16)
    m_avg = jnp.where(in_span, inv_len, 0.0).astype(jnp.bfloat16)
    m_s = (col == s).astype(jnp.bfloat16)
    m_e = (col == e - 1).astype(jnp.bfloat16)
    dn = (((0,), (1,)), ((), ()))
    o_ref[0, :, 0:d] = jax.lax.dot_general(
        m_s, fb, dn, preferred_element_type=jnp.float32
    )
    o_ref[0, :, d : 2 * d] = jax.lax.dot_general(
        m_avg, fb, dn, preferred_element_type=jnp.float32
    )
    o_ref[0, :, 2 * d : 3 * d] = jax.lax.dot_general(
        m_e, fb, dn, preferred_element_type=jnp.float32
    )


@jax.jit
def kernel(features, tois):
    b, d, t_len = features.shape
    n = tois.shape[1]
    nh = n // 2
    out = pl.pallas_call(
        functools.partial(_toi_tc_kernel, d=d, t_len=t_len, nh=nh),
        grid=(b, 2),
        in_specs=[
            pl.BlockSpec((1, 1, n), lambda i, j: (i, 0, 0)),
            pl.BlockSpec((1, 1, n), lambda i, j: (i, 0, 0)),
            pl.BlockSpec((1, d, t_len), lambda i, j: (i, 0, 0)),
        ],
        out_specs=pl.BlockSpec((1, nh, 3 * d), lambda i, j: (2 * i + j, 0, 0)),
        out_shape=jax.ShapeDtypeStruct((2 * b, nh, 3 * d), jnp.float32),
    )(
        tois[:, :, 0].reshape(b, 1, n),
        tois[:, :, 1].reshape(b, 1, n),
        features,
    )
    offsets = jnp.arange(1, b + 1, dtype=jnp.int32) * np.int32(n)
    return out.reshape(b * n, 3 * d), offsets
